# baseline (speedup 1.0000x reference)
import jax
import jax.numpy as jnp
from jax.experimental import pallas as pl

NUM_TYPES = 119
N_NODES = 100000
OUT_COLS = NUM_TYPES + 1

ROWS_PER_BLOCK = 2000
GRID = N_NODES // ROWS_PER_BLOCK


def _body(at_ref, spin_ref, out_ref):
    t = at_ref[:]
    lane = jax.lax.broadcasted_iota(jnp.int32, (ROWS_PER_BLOCK, OUT_COLS), 1)
    one_hot = (lane == t).astype(jnp.float32)
    sp = spin_ref[:]
    s = jnp.sum(sp * sp, axis=1, keepdims=True)
    norm = jnp.sqrt(s)
    d = jnp.maximum(norm, 1e-12)
    sn = s / (d * d)
    out_ref[:] = jnp.where(lane == NUM_TYPES, sn, one_hot)


def kernel(atom_type, pos, spin):
    del pos
    node_attrs = pl.pallas_call(
        _body,
        grid=(GRID,),
        in_specs=[
            pl.BlockSpec((ROWS_PER_BLOCK, 1), lambda i: (i, 0)),
            pl.BlockSpec((ROWS_PER_BLOCK, 3), lambda i: (i, 0)),
        ],
        out_specs=pl.BlockSpec((ROWS_PER_BLOCK, OUT_COLS), lambda i: (i, 0)),
        out_shape=jax.ShapeDtypeStruct((N_NODES, OUT_COLS), jnp.float32),
    )(atom_type, spin)
    return (node_attrs, node_attrs, spin)
